# baseline (speedup 1.0000x reference)
import jax
import jax.numpy as jnp
from jax import lax
from jax.experimental import pallas as pl
from jax.experimental.pallas import tpu as pltpu
from jax.experimental.pallas import tpu_sc as plsc

N = 10000
E = 320000
D_IN = 128
H = 256
OUT = 64
EPS = 1e-5

HH = H // 2
NTILES = 16
CHUNK = 128
NCHUNK = 160
G = 40
PT = NCHUNK * CHUNK
EP = NTILES * PT
ZROW = 2 * N
ROWCHUNK = 16
NROWCHUNK = N // ROWCHUNK
BN = 400
GRID = N // BN


def _sc_body(w2, src3, dst3, zrows, out, acc, src_v, dst_v, rows0, rows1,
             sem0, sem1):
    c = lax.axis_index("c")
    s = lax.axis_index("s")
    for t in range(NROWCHUNK // NTILES + 1):
        k = s + NTILES * t

        @pl.when(k < NROWCHUNK)
        def _():
            pltpu.sync_copy(zrows, acc.at[pl.ds(k * ROWCHUNK, ROWCHUNK)])

    plsc.subcore_barrier()

    def group(g, carry):
        pltpu.sync_copy(src3.at[c, s, pl.ds(g * G, G)], src_v)
        pltpu.sync_copy(dst3.at[s, pl.ds(g * G, G)], dst_v)

        pltpu.async_copy(w2.at[src_v.at[0]], rows0, sem0)
        pltpu.async_copy(w2.at[src_v.at[1]], rows1, sem1)

        def pair(i, carry2):
            j0 = 2 * i
            for j, rows, sem in ((j0, rows0, sem0), (j0 + 1, rows1, sem1)):
                pltpu.make_async_copy(w2.at[src_v.at[j]], rows, sem).wait()
                pltpu.sync_copy(rows, acc.at[dst_v.at[j]], add=True)

                @pl.when(j + 2 < G)
                def _():
                    pltpu.async_copy(w2.at[src_v.at[j + 2]], rows, sem)

            return carry2

        lax.fori_loop(0, G // 2, pair, 0)
        return carry

    lax.fori_loop(0, NCHUNK // G, group, 0)

    plsc.subcore_barrier()

    for t in range(NROWCHUNK // NTILES + 1):
        k = s + NTILES * t

        @pl.when(k < NROWCHUNK)
        def _():
            pltpu.sync_copy(
                acc.at[pl.ds(k * ROWCHUNK, ROWCHUNK)],
                out.at[c, pl.ds(k * ROWCHUNK, ROWCHUNK)],
            )


_sc_call = pl.kernel(
    _sc_body,
    out_type=jax.ShapeDtypeStruct((2, N, HH), jnp.float32),
    mesh=plsc.VectorSubcoreMesh(core_axis_name="c", subcore_axis_name="s"),
    scratch_types=[
        pltpu.VMEM_SHARED((N, HH), jnp.float32),
        pltpu.VMEM((G, CHUNK), jnp.int32),
        pltpu.VMEM((G, CHUNK), jnp.int32),
        pltpu.VMEM((CHUNK, HH), jnp.float32),
        pltpu.VMEM((CHUNK, HH), jnp.float32),
        pltpu.SemaphoreType.DMA,
        pltpu.SemaphoreType.DMA,
    ],
    compiler_params=pltpu.CompilerParams(use_tc_tiling_on_sc=False),
)


def _tc_body(seg, x, b_edge, g_en, be_en, W_em, b_em, W_n0, b_n0, g_nn,
             be_nn, W_n1, b_n1, W_c1, b_c1, W_c2, b_c2, W_f0, b_f0, g_fn,
             be_fn, W_f1, b_f1, out):
    inv = 1.0 / jnp.sqrt(jnp.float32(1.0 + EPS))
    s = jnp.concatenate([seg[0], seg[1]], axis=-1) + b_edge[...]
    s = jnp.maximum(s, 0.0)
    s = s * (inv * g_en[...]) + be_en[...]
    o = jnp.dot(s, W_em[...], preferred_element_type=jnp.float32) + b_em[...]
    o = o + jnp.dot(o, W_c1[...], preferred_element_type=jnp.float32) + b_c1[...]
    h = jnp.dot(x[...], W_n0[...], preferred_element_type=jnp.float32) + b_n0[...]
    h = jnp.maximum(h, 0.0)
    h = h * (inv * g_nn[...]) + be_nn[...]
    h2 = jnp.dot(h, W_n1[...], preferred_element_type=jnp.float32) + b_n1[...]
    o = o + h2 + jnp.dot(h2, W_c2[...], preferred_element_type=jnp.float32) + b_c2[...]
    o = jnp.maximum(o, 0.0)
    f = jnp.dot(o, W_f0[...], preferred_element_type=jnp.float32) + b_f0[...]
    f = jnp.maximum(f, 0.0)
    f = f * (inv * g_fn[...]) + be_fn[...]
    out[...] = jnp.dot(f, W_f1[...], preferred_element_type=jnp.float32) + b_f1[...]


def _full(shape):
    return pl.BlockSpec(shape, lambda i: (0,) * len(shape))


_tc_call = pl.pallas_call(
    _tc_body,
    grid=(GRID,),
    in_specs=[
        pl.BlockSpec((2, BN, HH), lambda i: (0, i, 0)),
        pl.BlockSpec((BN, D_IN), lambda i: (i, 0)),
        _full((1, H)),
        _full((1, H)), _full((1, H)),
        _full((H, H)), _full((1, H)),
        _full((D_IN, H)), _full((1, H)),
        _full((1, H)), _full((1, H)),
        _full((H, H)), _full((1, H)),
        _full((H, H)), _full((1, H)),
        _full((H, H)), _full((1, H)),
        _full((H, H)), _full((1, H)),
        _full((1, H)), _full((1, H)),
        _full((H, OUT)), _full((1, OUT)),
    ],
    out_specs=pl.BlockSpec((BN, OUT), lambda i: (i, 0)),
    out_shape=jax.ShapeDtypeStruct((N, OUT), jnp.float32),
)


def kernel(x, edge_index, W_edge, b_edge, g_en, be_en, W_em, b_em, W_n0,
           b_n0, g_nn, be_nn, W_n1, b_n1, W_c1, b_c1, W_c2, b_c2, W_f0,
           b_f0, g_fn, be_fn, W_f1, b_f1):
    src = edge_index[0]
    dst = edge_index[1]
    w2 = jnp.concatenate(
        [W_edge[:, :HH], W_edge[:, HH:], jnp.zeros((8, HH), jnp.float32)],
        axis=0)
    padi = jnp.full((EP - E,), ZROW, jnp.int32)
    src3 = jnp.stack([
        jnp.concatenate([src, padi]),
        jnp.concatenate([src + N, padi]),
    ]).reshape(2, NTILES, NCHUNK, CHUNK)
    padd = (jnp.arange(EP - E, dtype=jnp.int32) * 7) % N
    dst3 = jnp.concatenate([dst, padd]).reshape(NTILES, NCHUNK, CHUNK)
    zrows = jnp.zeros((ROWCHUNK, HH), jnp.float32)

    seg = _sc_call(w2, src3, dst3, zrows)

    r1 = lambda v: v.reshape(1, -1)
    return _tc_call(seg, x, r1(b_edge), r1(g_en), r1(be_en), W_em, r1(b_em),
                    W_n0, r1(b_n0), r1(g_nn), r1(be_nn), W_n1, r1(b_n1),
                    W_c1, r1(b_c1), W_c2, r1(b_c2), W_f0, r1(b_f0),
                    r1(g_fn), r1(be_fn), W_f1, r1(b_f1))


# speedup vs baseline: 2.7104x; 2.7104x over previous
import jax
import jax.numpy as jnp
from jax import lax
from jax.experimental import pallas as pl
from jax.experimental.pallas import tpu as pltpu
from jax.experimental.pallas import tpu_sc as plsc

N = 10000
E = 320000
D_IN = 128
H = 256
OUT = 64
EPS = 1e-5

HH = H // 2
NTILES = 16
CHUNK = 80
NCHUNK = E // NTILES // CHUNK
G = 50
ROWCHUNK = 16
NROWCHUNK = N // ROWCHUNK
BN = 400
GRID = N // BN


def _sc_body(w2, src3, dst3, zrows, out, acc, src_v, dst_v, rows0, rows1,
             rows2, rows3, sem0, sem1, sem2, sem3):
    rows = (rows0, rows1, rows2, rows3)
    sems = (sem0, sem1, sem2, sem3)
    c = lax.axis_index("c")
    s = lax.axis_index("s")
    for t in range(NROWCHUNK // NTILES + 1):
        k = s + NTILES * t

        @pl.when(k < NROWCHUNK)
        def _():
            pltpu.sync_copy(zrows, acc.at[pl.ds(k * ROWCHUNK, ROWCHUNK)])

    plsc.subcore_barrier()

    def group(g, carry):
        pltpu.sync_copy(src3.at[c, s, pl.ds(g * G, G)], src_v)
        pltpu.sync_copy(dst3.at[s, pl.ds(g * G, G)], dst_v)

        for b in range(4):
            pltpu.async_copy(w2.at[src_v.at[b]], rows[b], sems[b])

        def quad(i, carry2):
            j0 = 4 * i
            for b in range(4):
                j = j0 + b
                pltpu.make_async_copy(
                    w2.at[src_v.at[j]], rows[b], sems[b]).wait()
                pltpu.sync_copy(rows[b], acc.at[dst_v.at[j]], add=True)

                @pl.when(j + 4 < G)
                def _():
                    pltpu.async_copy(w2.at[src_v.at[j + 4]], rows[b], sems[b])

            return carry2

        lax.fori_loop(0, G // 4, quad, 0)
        for b in range(G - 4 * (G // 4)):
            j = 4 * (G // 4) + b
            pltpu.make_async_copy(w2.at[src_v.at[j]], rows[b], sems[b]).wait()
            pltpu.sync_copy(rows[b], acc.at[dst_v.at[j]], add=True)
        return carry

    lax.fori_loop(0, NCHUNK // G, group, 0)

    plsc.subcore_barrier()

    for t in range(NROWCHUNK // NTILES + 1):
        k = s + NTILES * t

        @pl.when(k < NROWCHUNK)
        def _():
            pltpu.sync_copy(
                acc.at[pl.ds(k * ROWCHUNK, ROWCHUNK)],
                out.at[c, pl.ds(k * ROWCHUNK, ROWCHUNK)],
            )


_sc_call = pl.kernel(
    _sc_body,
    out_type=jax.ShapeDtypeStruct((2, N, HH), jnp.float32),
    mesh=plsc.VectorSubcoreMesh(core_axis_name="c", subcore_axis_name="s"),
    scratch_types=[
        pltpu.VMEM_SHARED((N, HH), jnp.float32),
        pltpu.VMEM((G, CHUNK), jnp.int32),
        pltpu.VMEM((G, CHUNK), jnp.int32),
        pltpu.VMEM((CHUNK, HH), jnp.float32),
        pltpu.VMEM((CHUNK, HH), jnp.float32),
        pltpu.VMEM((CHUNK, HH), jnp.float32),
        pltpu.VMEM((CHUNK, HH), jnp.float32),
        pltpu.SemaphoreType.DMA,
        pltpu.SemaphoreType.DMA,
        pltpu.SemaphoreType.DMA,
        pltpu.SemaphoreType.DMA,
    ],
    compiler_params=pltpu.CompilerParams(use_tc_tiling_on_sc=False),
)


def _tc_body(seg, x, b_edge, g_en, be_en, W_em, b_em, W_n0, b_n0, g_nn,
             be_nn, W_n1, b_n1, W_c1, b_c1, W_c2, b_c2, W_f0, b_f0, g_fn,
             be_fn, W_f1, b_f1, out):
    inv = 1.0 / jnp.sqrt(jnp.float32(1.0 + EPS))
    s = jnp.concatenate([seg[0], seg[1]], axis=-1) + b_edge[...]
    s = jnp.maximum(s, 0.0)
    s = s * (inv * g_en[...]) + be_en[...]
    o = jnp.dot(s, W_em[...], preferred_element_type=jnp.float32) + b_em[...]
    o = o + jnp.dot(o, W_c1[...], preferred_element_type=jnp.float32) + b_c1[...]
    h = jnp.dot(x[...], W_n0[...], preferred_element_type=jnp.float32) + b_n0[...]
    h = jnp.maximum(h, 0.0)
    h = h * (inv * g_nn[...]) + be_nn[...]
    h2 = jnp.dot(h, W_n1[...], preferred_element_type=jnp.float32) + b_n1[...]
    o = o + h2 + jnp.dot(h2, W_c2[...], preferred_element_type=jnp.float32) + b_c2[...]
    o = jnp.maximum(o, 0.0)
    f = jnp.dot(o, W_f0[...], preferred_element_type=jnp.float32) + b_f0[...]
    f = jnp.maximum(f, 0.0)
    f = f * (inv * g_fn[...]) + be_fn[...]
    out[...] = jnp.dot(f, W_f1[...], preferred_element_type=jnp.float32) + b_f1[...]


def _full(shape):
    return pl.BlockSpec(shape, lambda i: (0,) * len(shape))


_tc_call = pl.pallas_call(
    _tc_body,
    grid=(GRID,),
    in_specs=[
        pl.BlockSpec((2, BN, HH), lambda i: (0, i, 0)),
        pl.BlockSpec((BN, D_IN), lambda i: (i, 0)),
        _full((1, H)),
        _full((1, H)), _full((1, H)),
        _full((H, H)), _full((1, H)),
        _full((D_IN, H)), _full((1, H)),
        _full((1, H)), _full((1, H)),
        _full((H, H)), _full((1, H)),
        _full((H, H)), _full((1, H)),
        _full((H, H)), _full((1, H)),
        _full((H, H)), _full((1, H)),
        _full((1, H)), _full((1, H)),
        _full((H, OUT)), _full((1, OUT)),
    ],
    out_specs=pl.BlockSpec((BN, OUT), lambda i: (i, 0)),
    out_shape=jax.ShapeDtypeStruct((N, OUT), jnp.float32),
)


def kernel(x, edge_index, W_edge, b_edge, g_en, be_en, W_em, b_em, W_n0,
           b_n0, g_nn, be_nn, W_n1, b_n1, W_c1, b_c1, W_c2, b_c2, W_f0,
           b_f0, g_fn, be_fn, W_f1, b_f1):
    src = edge_index[0]
    dst = edge_index[1]
    w2 = jnp.concatenate([W_edge[:, :HH], W_edge[:, HH:]], axis=0)
    src3 = jnp.stack([src, src + N]).reshape(2, NTILES, NCHUNK, CHUNK)
    dst3 = dst.reshape(NTILES, NCHUNK, CHUNK)
    zrows = jnp.zeros((ROWCHUNK, HH), jnp.float32)

    seg = _sc_call(w2, src3, dst3, zrows)

    r1 = lambda v: v.reshape(1, -1)
    return _tc_call(seg, x, r1(b_edge), r1(g_en), r1(be_en), W_em, r1(b_em),
                    W_n0, r1(b_n0), r1(g_nn), r1(be_nn), W_n1, r1(b_n1),
                    W_c1, r1(b_c1), W_c2, r1(b_c2), W_f0, r1(b_f0),
                    r1(g_fn), r1(be_fn), W_f1, r1(b_f1))


# speedup vs baseline: 3.4622x; 1.2774x over previous
import jax
import jax.numpy as jnp
from jax import lax
from jax.experimental import pallas as pl
from jax.experimental.pallas import tpu as pltpu
from jax.experimental.pallas import tpu_sc as plsc

N = 10000
E = 320000
D_IN = 128
H = 256
OUT = 64
EPS = 1e-5

HH = H // 2
NTILES = 16
CHUNK = 80
NCHUNK = E // NTILES // CHUNK
G = 50
STRIPE = N // NTILES
BN = 400
GRID = N // BN


def _sc_body(w2, src3, dst3, zrows, out, acc, src_v, dst_v, rows0, rows1,
             rows2, rows3, sem0, sem1, sem2, sem3):
    rows = (rows0, rows1, rows2, rows3)
    sems = (sem0, sem1, sem2, sem3)
    c = lax.axis_index("c")
    s = lax.axis_index("s")
    pltpu.sync_copy(zrows, acc.at[pl.ds(s * STRIPE, STRIPE)])
    plsc.subcore_barrier()

    def group(g, carry):
        pltpu.sync_copy(src3.at[c, s, pl.ds(g * G, G)], src_v)
        pltpu.sync_copy(dst3.at[s, pl.ds(g * G, G)], dst_v)

        for b in range(4):
            pltpu.async_copy(w2.at[src_v.at[b]], rows[b], sems[b])

        def quad(i, carry2):
            j0 = 4 * i
            for b in range(4):
                j = j0 + b
                pltpu.make_async_copy(
                    w2.at[src_v.at[j]], rows[b], sems[b]).wait()
                pltpu.sync_copy(rows[b], acc.at[dst_v.at[j]], add=True)

                @pl.when(j + 4 < G)
                def _():
                    pltpu.async_copy(w2.at[src_v.at[j + 4]], rows[b], sems[b])

            return carry2

        lax.fori_loop(0, G // 4, quad, 0)
        for b in range(G - 4 * (G // 4)):
            j = 4 * (G // 4) + b
            pltpu.make_async_copy(w2.at[src_v.at[j]], rows[b], sems[b]).wait()
            pltpu.sync_copy(rows[b], acc.at[dst_v.at[j]], add=True)
        return carry

    lax.fori_loop(0, NCHUNK // G, group, 0)

    plsc.subcore_barrier()

    pltpu.sync_copy(
        acc.at[pl.ds(s * STRIPE, STRIPE)],
        out.at[c, pl.ds(s * STRIPE, STRIPE)],
    )


_sc_call = pl.kernel(
    _sc_body,
    out_type=jax.ShapeDtypeStruct((2, N, HH), jnp.float32),
    mesh=plsc.VectorSubcoreMesh(core_axis_name="c", subcore_axis_name="s"),
    scratch_types=[
        pltpu.VMEM_SHARED((N, HH), jnp.float32),
        pltpu.VMEM((G, CHUNK), jnp.int32),
        pltpu.VMEM((G, CHUNK), jnp.int32),
        pltpu.VMEM((CHUNK, HH), jnp.float32),
        pltpu.VMEM((CHUNK, HH), jnp.float32),
        pltpu.VMEM((CHUNK, HH), jnp.float32),
        pltpu.VMEM((CHUNK, HH), jnp.float32),
        pltpu.SemaphoreType.DMA,
        pltpu.SemaphoreType.DMA,
        pltpu.SemaphoreType.DMA,
        pltpu.SemaphoreType.DMA,
    ],
    compiler_params=pltpu.CompilerParams(use_tc_tiling_on_sc=False),
)


def _tc_body(seg, x, b_edge, g_en, be_en, W_em, b_em, W_n0, b_n0, g_nn,
             be_nn, W_n1, b_n1, W_c1, b_c1, W_c2, b_c2, W_f0, b_f0, g_fn,
             be_fn, W_f1, b_f1, out):
    inv = 1.0 / jnp.sqrt(jnp.float32(1.0 + EPS))
    s = jnp.concatenate([seg[0], seg[1]], axis=-1) + b_edge[...]
    s = jnp.maximum(s, 0.0)
    s = s * (inv * g_en[...]) + be_en[...]
    o = jnp.dot(s, W_em[...], preferred_element_type=jnp.float32) + b_em[...]
    o = o + jnp.dot(o, W_c1[...], preferred_element_type=jnp.float32) + b_c1[...]
    h = jnp.dot(x[...], W_n0[...], preferred_element_type=jnp.float32) + b_n0[...]
    h = jnp.maximum(h, 0.0)
    h = h * (inv * g_nn[...]) + be_nn[...]
    h2 = jnp.dot(h, W_n1[...], preferred_element_type=jnp.float32) + b_n1[...]
    o = o + h2 + jnp.dot(h2, W_c2[...], preferred_element_type=jnp.float32) + b_c2[...]
    o = jnp.maximum(o, 0.0)
    f = jnp.dot(o, W_f0[...], preferred_element_type=jnp.float32) + b_f0[...]
    f = jnp.maximum(f, 0.0)
    f = f * (inv * g_fn[...]) + be_fn[...]
    out[...] = jnp.dot(f, W_f1[...], preferred_element_type=jnp.float32) + b_f1[...]


def _full(shape):
    return pl.BlockSpec(shape, lambda i: (0,) * len(shape))


_tc_call = pl.pallas_call(
    _tc_body,
    grid=(GRID,),
    in_specs=[
        pl.BlockSpec((2, BN, HH), lambda i: (0, i, 0)),
        pl.BlockSpec((BN, D_IN), lambda i: (i, 0)),
        _full((1, H)),
        _full((1, H)), _full((1, H)),
        _full((H, H)), _full((1, H)),
        _full((D_IN, H)), _full((1, H)),
        _full((1, H)), _full((1, H)),
        _full((H, H)), _full((1, H)),
        _full((H, H)), _full((1, H)),
        _full((H, H)), _full((1, H)),
        _full((H, H)), _full((1, H)),
        _full((1, H)), _full((1, H)),
        _full((H, OUT)), _full((1, OUT)),
    ],
    out_specs=pl.BlockSpec((BN, OUT), lambda i: (i, 0)),
    out_shape=jax.ShapeDtypeStruct((N, OUT), jnp.float32),
)


def kernel(x, edge_index, W_edge, b_edge, g_en, be_en, W_em, b_em, W_n0,
           b_n0, g_nn, be_nn, W_n1, b_n1, W_c1, b_c1, W_c2, b_c2, W_f0,
           b_f0, g_fn, be_fn, W_f1, b_f1):
    src = edge_index[0]
    dst = edge_index[1]
    w2 = W_edge.reshape(2 * N, HH)
    src3 = jnp.stack([src * 2, src * 2 + 1]).reshape(2, NTILES, NCHUNK, CHUNK)
    dst3 = dst.reshape(NTILES, NCHUNK, CHUNK)
    zrows = jnp.zeros((STRIPE, HH), jnp.float32)

    seg = _sc_call(w2, src3, dst3, zrows)

    r1 = lambda v: v.reshape(1, -1)
    return _tc_call(seg, x, r1(b_edge), r1(g_en), r1(be_en), W_em, r1(b_em),
                    W_n0, r1(b_n0), r1(g_nn), r1(be_nn), W_n1, r1(b_n1),
                    W_c1, r1(b_c1), W_c2, r1(b_c2), W_f0, r1(b_f0),
                    r1(g_fn), r1(be_fn), W_f1, r1(b_f1))


# speedup vs baseline: 3.5949x; 1.0383x over previous
import jax
import jax.numpy as jnp
from jax import lax
from jax.experimental import pallas as pl
from jax.experimental.pallas import tpu as pltpu
from jax.experimental.pallas import tpu_sc as plsc

N = 10000
E = 320000
D_IN = 128
H = 256
OUT = 64
EPS = 1e-5

HH = H // 2
NTILES = 16
CHUNK = 80
NCHUNK = E // NTILES // CHUNK
G = 50
STRIPE = N // NTILES
BN = 1000
GRID = N // BN


def _sc_body(w2, src3, dst3, zrows, out, acc, src_v, dst_v, rows0, rows1,
             rows2, rows3, sem0, sem1, sem2, sem3):
    rows = (rows0, rows1, rows2, rows3)
    sems = (sem0, sem1, sem2, sem3)
    c = lax.axis_index("c")
    s = lax.axis_index("s")
    pltpu.sync_copy(zrows, acc.at[pl.ds(s * STRIPE, STRIPE)])
    plsc.subcore_barrier()

    def group(g, carry):
        pltpu.sync_copy(src3.at[c, s, pl.ds(g * G, G)], src_v)
        pltpu.sync_copy(dst3.at[s, pl.ds(g * G, G)], dst_v)

        for b in range(4):
            pltpu.async_copy(w2.at[src_v.at[b]], rows[b], sems[b])

        def quad(i, carry2):
            j0 = 4 * i
            for b in range(4):
                j = j0 + b
                pltpu.make_async_copy(
                    w2.at[src_v.at[j]], rows[b], sems[b]).wait()
                pltpu.sync_copy(rows[b], acc.at[dst_v.at[j]], add=True)

                @pl.when(j + 4 < G)
                def _():
                    pltpu.async_copy(w2.at[src_v.at[j + 4]], rows[b], sems[b])

            return carry2

        lax.fori_loop(0, G // 4, quad, 0)
        for b in range(G - 4 * (G // 4)):
            j = 4 * (G // 4) + b
            pltpu.make_async_copy(w2.at[src_v.at[j]], rows[b], sems[b]).wait()
            pltpu.sync_copy(rows[b], acc.at[dst_v.at[j]], add=True)
        return carry

    lax.fori_loop(0, NCHUNK // G, group, 0)

    plsc.subcore_barrier()

    pltpu.sync_copy(
        acc.at[pl.ds(s * STRIPE, STRIPE)],
        out.at[c, pl.ds(s * STRIPE, STRIPE)],
    )


_sc_call = pl.kernel(
    _sc_body,
    out_type=jax.ShapeDtypeStruct((2, N, HH), jnp.float32),
    mesh=plsc.VectorSubcoreMesh(core_axis_name="c", subcore_axis_name="s"),
    scratch_types=[
        pltpu.VMEM_SHARED((N, HH), jnp.float32),
        pltpu.VMEM((G, CHUNK), jnp.int32),
        pltpu.VMEM((G, CHUNK), jnp.int32),
        pltpu.VMEM((CHUNK, HH), jnp.float32),
        pltpu.VMEM((CHUNK, HH), jnp.float32),
        pltpu.VMEM((CHUNK, HH), jnp.float32),
        pltpu.VMEM((CHUNK, HH), jnp.float32),
        pltpu.SemaphoreType.DMA,
        pltpu.SemaphoreType.DMA,
        pltpu.SemaphoreType.DMA,
        pltpu.SemaphoreType.DMA,
    ],
    compiler_params=pltpu.CompilerParams(use_tc_tiling_on_sc=False),
)


def _tc_body(seg, x, b_edge, g_en, be_en, W_em, b_em, W_n0, b_n0, g_nn,
             be_nn, W_n1, b_n1, W_c1, b_c1, W_c2, b_c2, W_f0, b_f0, g_fn,
             be_fn, W_f1, b_f1, out):
    inv = 1.0 / jnp.sqrt(jnp.float32(1.0 + EPS))
    s = jnp.concatenate([seg[0], seg[1]], axis=-1) + b_edge[...]
    s = jnp.maximum(s, 0.0)
    s = s * (inv * g_en[...]) + be_en[...]
    o = jnp.dot(s, W_em[...], preferred_element_type=jnp.float32) + b_em[...]
    o = o + jnp.dot(o, W_c1[...], preferred_element_type=jnp.float32) + b_c1[...]
    h = jnp.dot(x[...], W_n0[...], preferred_element_type=jnp.float32) + b_n0[...]
    h = jnp.maximum(h, 0.0)
    h = h * (inv * g_nn[...]) + be_nn[...]
    h2 = jnp.dot(h, W_n1[...], preferred_element_type=jnp.float32) + b_n1[...]
    o = o + h2 + jnp.dot(h2, W_c2[...], preferred_element_type=jnp.float32) + b_c2[...]
    o = jnp.maximum(o, 0.0)
    f = jnp.dot(o, W_f0[...], preferred_element_type=jnp.float32) + b_f0[...]
    f = jnp.maximum(f, 0.0)
    f = f * (inv * g_fn[...]) + be_fn[...]
    out[...] = jnp.dot(f, W_f1[...], preferred_element_type=jnp.float32) + b_f1[...]


def _full(shape):
    return pl.BlockSpec(shape, lambda i: (0,) * len(shape))


_tc_call = pl.pallas_call(
    _tc_body,
    grid=(GRID,),
    in_specs=[
        pl.BlockSpec((2, BN, HH), lambda i: (0, i, 0)),
        pl.BlockSpec((BN, D_IN), lambda i: (i, 0)),
        _full((1, H)),
        _full((1, H)), _full((1, H)),
        _full((H, H)), _full((1, H)),
        _full((D_IN, H)), _full((1, H)),
        _full((1, H)), _full((1, H)),
        _full((H, H)), _full((1, H)),
        _full((H, H)), _full((1, H)),
        _full((H, H)), _full((1, H)),
        _full((H, H)), _full((1, H)),
        _full((1, H)), _full((1, H)),
        _full((H, OUT)), _full((1, OUT)),
    ],
    out_specs=pl.BlockSpec((BN, OUT), lambda i: (i, 0)),
    out_shape=jax.ShapeDtypeStruct((N, OUT), jnp.float32),
)


def kernel(x, edge_index, W_edge, b_edge, g_en, be_en, W_em, b_em, W_n0,
           b_n0, g_nn, be_nn, W_n1, b_n1, W_c1, b_c1, W_c2, b_c2, W_f0,
           b_f0, g_fn, be_fn, W_f1, b_f1):
    src = edge_index[0]
    dst = edge_index[1]
    w2 = W_edge.reshape(2 * N, HH)
    src3 = jnp.stack([src * 2, src * 2 + 1]).reshape(2, NTILES, NCHUNK, CHUNK)
    dst3 = dst.reshape(NTILES, NCHUNK, CHUNK)
    zrows = jnp.zeros((STRIPE, HH), jnp.float32)

    seg = _sc_call(w2, src3, dst3, zrows)

    r1 = lambda v: v.reshape(1, -1)
    return _tc_call(seg, x, r1(b_edge), r1(g_en), r1(be_en), W_em, r1(b_em),
                    W_n0, r1(b_n0), r1(g_nn), r1(be_nn), W_n1, r1(b_n1),
                    W_c1, r1(b_c1), W_c2, r1(b_c2), W_f0, r1(b_f0),
                    r1(g_fn), r1(be_fn), W_f1, r1(b_f1))


# speedup vs baseline: 3.6516x; 1.0158x over previous
import jax
import jax.numpy as jnp
from jax import lax
from jax.experimental import pallas as pl
from jax.experimental.pallas import tpu as pltpu
from jax.experimental.pallas import tpu_sc as plsc

N = 10000
E = 320000
D_IN = 128
H = 256
OUT = 64
EPS = 1e-5

HH = H // 2
NTILES = 16
CHUNK = 80
NCHUNK = E // NTILES // CHUNK
G = 50
STRIPE = N // NTILES
BN = 1000
GRID = N // BN


def _sc_body(w2, src3, dst3, zrows, out, acc, src_v, dst_v, rows0, rows1,
             rows2, rows3, sem0, sem1, sem2, sem3):
    rows = (rows0, rows1, rows2, rows3)
    sems = (sem0, sem1, sem2, sem3)
    c = lax.axis_index("c")
    s = lax.axis_index("s")
    pltpu.sync_copy(zrows, acc.at[pl.ds(s * STRIPE, STRIPE)])
    plsc.subcore_barrier()

    def group(g, carry):
        pltpu.sync_copy(src3.at[c, s, pl.ds(g * G, G)], src_v)
        pltpu.sync_copy(dst3.at[s, pl.ds(g * G, G)], dst_v)

        for b in range(4):
            pltpu.async_copy(w2.at[src_v.at[b]], rows[b], sems[b])

        def quad(i, carry2):
            j0 = 4 * i
            for b in range(4):
                j = j0 + b
                pltpu.make_async_copy(
                    w2.at[src_v.at[j]], rows[b], sems[b]).wait()
                pltpu.sync_copy(rows[b], acc.at[dst_v.at[j]], add=True)

                @pl.when(j + 4 < G)
                def _():
                    pltpu.async_copy(w2.at[src_v.at[j + 4]], rows[b], sems[b])

            return carry2

        lax.fori_loop(0, G // 4, quad, 0)
        for b in range(G - 4 * (G // 4)):
            j = 4 * (G // 4) + b
            pltpu.make_async_copy(w2.at[src_v.at[j]], rows[b], sems[b]).wait()
            pltpu.sync_copy(rows[b], acc.at[dst_v.at[j]], add=True)
        return carry

    lax.fori_loop(0, NCHUNK // G, group, 0)

    plsc.subcore_barrier()

    pltpu.sync_copy(
        acc.at[pl.ds(s * STRIPE, STRIPE)],
        out.at[c, pl.ds(s * STRIPE, STRIPE)],
    )


_sc_call = pl.kernel(
    _sc_body,
    out_type=jax.ShapeDtypeStruct((2, N, HH), jnp.float32),
    mesh=plsc.VectorSubcoreMesh(core_axis_name="c", subcore_axis_name="s"),
    scratch_types=[
        pltpu.VMEM_SHARED((N, HH), jnp.float32),
        pltpu.VMEM((G, CHUNK), jnp.int32),
        pltpu.VMEM((G, CHUNK), jnp.int32),
        pltpu.VMEM((CHUNK, HH), jnp.float32),
        pltpu.VMEM((CHUNK, HH), jnp.float32),
        pltpu.VMEM((CHUNK, HH), jnp.float32),
        pltpu.VMEM((CHUNK, HH), jnp.float32),
        pltpu.SemaphoreType.DMA,
        pltpu.SemaphoreType.DMA,
        pltpu.SemaphoreType.DMA,
        pltpu.SemaphoreType.DMA,
    ],
    compiler_params=pltpu.CompilerParams(use_tc_tiling_on_sc=False),
)


def _tc_body(seg, x, b_edge, g_en, be_en, W_em, b_em, W_n0, b_n0, g_nn,
             be_nn, W_n1, b_n1, W_c1, b_c1, W_c2, b_c2, W_f0, b_f0, g_fn,
             be_fn, W_f1, b_f1, out):
    inv = 1.0 / jnp.sqrt(jnp.float32(1.0 + EPS))
    s = jnp.concatenate([seg[0], seg[1]], axis=-1) + b_edge[...]
    s = jnp.maximum(s, 0.0)
    s = s * (inv * g_en[...]) + be_en[...]
    o = jnp.dot(s, W_em[...], preferred_element_type=jnp.float32) + b_em[...]
    o = o + jnp.dot(o, W_c1[...], preferred_element_type=jnp.float32) + b_c1[...]
    h = jnp.dot(x[...], W_n0[...], preferred_element_type=jnp.float32) + b_n0[...]
    h = jnp.maximum(h, 0.0)
    h = h * (inv * g_nn[...]) + be_nn[...]
    h2 = jnp.dot(h, W_n1[...], preferred_element_type=jnp.float32) + b_n1[...]
    o = o + h2 + jnp.dot(h2, W_c2[...], preferred_element_type=jnp.float32) + b_c2[...]
    o = jnp.maximum(o, 0.0)
    f = jnp.dot(o, W_f0[...], preferred_element_type=jnp.float32) + b_f0[...]
    f = jnp.maximum(f, 0.0)
    f = f * (inv * g_fn[...]) + be_fn[...]
    out[...] = jnp.dot(f, W_f1[...], preferred_element_type=jnp.float32) + b_f1[...]


def _full(shape):
    return pl.BlockSpec(shape, lambda i: (0,) * len(shape))


_tc_call = pl.pallas_call(
    _tc_body,
    grid=(GRID,),
    in_specs=[
        pl.BlockSpec((2, BN, HH), lambda i: (0, i, 0)),
        pl.BlockSpec((BN, D_IN), lambda i: (i, 0)),
        _full((1, H)),
        _full((1, H)), _full((1, H)),
        _full((H, H)), _full((1, H)),
        _full((D_IN, H)), _full((1, H)),
        _full((1, H)), _full((1, H)),
        _full((H, H)), _full((1, H)),
        _full((H, H)), _full((1, H)),
        _full((H, H)), _full((1, H)),
        _full((H, H)), _full((1, H)),
        _full((1, H)), _full((1, H)),
        _full((H, OUT)), _full((1, OUT)),
    ],
    out_specs=pl.BlockSpec((BN, OUT), lambda i: (i, 0)),
    out_shape=jax.ShapeDtypeStruct((N, OUT), jnp.float32),
)


def kernel(x, edge_index, W_edge, b_edge, g_en, be_en, W_em, b_em, W_n0,
           b_n0, g_nn, be_nn, W_n1, b_n1, W_c1, b_c1, W_c2, b_c2, W_f0,
           b_f0, g_fn, be_fn, W_f1, b_f1):
    src = edge_index[0]
    dst = edge_index[1]
    w2 = W_edge.reshape(2 * N, HH)
    src3 = (jnp.broadcast_to(src * 2, (2, E))
            + jnp.array([[0], [1]], jnp.int32)
            ).reshape(2, NTILES, NCHUNK, CHUNK)
    dst3 = dst.reshape(NTILES, NCHUNK, CHUNK)
    zrows = jnp.zeros((STRIPE, HH), jnp.float32)

    seg = _sc_call(w2, src3, dst3, zrows)

    r1 = lambda v: v.reshape(1, -1)
    return _tc_call(seg, x, r1(b_edge), r1(g_en), r1(be_en), W_em, r1(b_em),
                    W_n0, r1(b_n0), r1(g_nn), r1(be_nn), W_n1, r1(b_n1),
                    W_c1, r1(b_c1), W_c2, r1(b_c2), W_f0, r1(b_f0),
                    r1(g_fn), r1(be_fn), W_f1, r1(b_f1))


# speedup vs baseline: 3.7357x; 1.0230x over previous
import jax
import jax.numpy as jnp
from jax import lax
from jax.experimental import pallas as pl
from jax.experimental.pallas import tpu as pltpu
from jax.experimental.pallas import tpu_sc as plsc

N = 10000
E = 320000
D_IN = 128
H = 256
OUT = 64
EPS = 1e-5

HH = H // 2
NTILES = 16
CHUNK = 80
NCHUNK = E // NTILES // CHUNK
G = 50
PT = NCHUNK * CHUNK
STRIPE = N // NTILES
BN = 1000
GRID = N // BN


def _sc_body(w2, src3, dst3, zrows, out, acc, src_v, dst_v, rows0, rows1,
             rows2, rows3, sem0, sem1, sem2, sem3):
    rows = (rows0, rows1, rows2, rows3)
    sems = (sem0, sem1, sem2, sem3)
    c = lax.axis_index("c")
    s = lax.axis_index("s")
    pltpu.sync_copy(zrows, acc.at[pl.ds(s * STRIPE, STRIPE)])
    plsc.subcore_barrier()

    def group(g, carry):
        pltpu.sync_copy(src3.at[pl.ds(s * PT + g * G * CHUNK, G * CHUNK)],
                        src_v)
        pltpu.sync_copy(dst3.at[s, pl.ds(g * G, G)], dst_v)

        def xform(i, carry3):
            v = src_v[pl.ds(i * 16, 16)]
            src_v[pl.ds(i * 16, 16)] = v * 2 + c
            return carry3

        lax.fori_loop(0, G * CHUNK // 16, xform, 0)

        for b in range(4):
            pltpu.async_copy(
                w2.at[src_v.at[pl.ds(b * CHUNK, CHUNK)]], rows[b], sems[b])

        def quad(i, carry2):
            j0 = 4 * i
            for b in range(4):
                j = j0 + b
                pltpu.make_async_copy(
                    w2.at[src_v.at[pl.ds(j * CHUNK, CHUNK)]],
                    rows[b], sems[b]).wait()
                pltpu.sync_copy(rows[b], acc.at[dst_v.at[j]], add=True)

                @pl.when(j + 4 < G)
                def _():
                    pltpu.async_copy(
                        w2.at[src_v.at[pl.ds((j + 4) * CHUNK, CHUNK)]],
                        rows[b], sems[b])

            return carry2

        lax.fori_loop(0, G // 4, quad, 0)
        for b in range(G - 4 * (G // 4)):
            j = 4 * (G // 4) + b
            pltpu.make_async_copy(
                w2.at[src_v.at[pl.ds(j * CHUNK, CHUNK)]],
                rows[b], sems[b]).wait()
            pltpu.sync_copy(rows[b], acc.at[dst_v.at[j]], add=True)
        return carry

    lax.fori_loop(0, NCHUNK // G, group, 0)

    plsc.subcore_barrier()

    pltpu.sync_copy(
        acc.at[pl.ds(s * STRIPE, STRIPE)],
        out.at[c, pl.ds(s * STRIPE, STRIPE)],
    )


_sc_call = pl.kernel(
    _sc_body,
    out_type=jax.ShapeDtypeStruct((2, N, HH), jnp.float32),
    mesh=plsc.VectorSubcoreMesh(core_axis_name="c", subcore_axis_name="s"),
    scratch_types=[
        pltpu.VMEM_SHARED((N, HH), jnp.float32),
        pltpu.VMEM((G * CHUNK,), jnp.int32),
        pltpu.VMEM((G, CHUNK), jnp.int32),
        pltpu.VMEM((CHUNK, HH), jnp.float32),
        pltpu.VMEM((CHUNK, HH), jnp.float32),
        pltpu.VMEM((CHUNK, HH), jnp.float32),
        pltpu.VMEM((CHUNK, HH), jnp.float32),
        pltpu.SemaphoreType.DMA,
        pltpu.SemaphoreType.DMA,
        pltpu.SemaphoreType.DMA,
        pltpu.SemaphoreType.DMA,
    ],
    compiler_params=pltpu.CompilerParams(use_tc_tiling_on_sc=False),
)


def _tc_body(seg, x, b_edge, g_en, be_en, W_em, b_em, W_n0, b_n0, g_nn,
             be_nn, W_n1, b_n1, W_c1, b_c1, W_c2, b_c2, W_f0, b_f0, g_fn,
             be_fn, W_f1, b_f1, out):
    inv = 1.0 / jnp.sqrt(jnp.float32(1.0 + EPS))
    s = jnp.concatenate([seg[0], seg[1]], axis=-1) + b_edge[...]
    s = jnp.maximum(s, 0.0)
    s = s * (inv * g_en[...]) + be_en[...]
    o = jnp.dot(s, W_em[...], preferred_element_type=jnp.float32) + b_em[...]
    o = o + jnp.dot(o, W_c1[...], preferred_element_type=jnp.float32) + b_c1[...]
    h = jnp.dot(x[...], W_n0[...], preferred_element_type=jnp.float32) + b_n0[...]
    h = jnp.maximum(h, 0.0)
    h = h * (inv * g_nn[...]) + be_nn[...]
    h2 = jnp.dot(h, W_n1[...], preferred_element_type=jnp.float32) + b_n1[...]
    o = o + h2 + jnp.dot(h2, W_c2[...], preferred_element_type=jnp.float32) + b_c2[...]
    o = jnp.maximum(o, 0.0)
    f = jnp.dot(o, W_f0[...], preferred_element_type=jnp.float32) + b_f0[...]
    f = jnp.maximum(f, 0.0)
    f = f * (inv * g_fn[...]) + be_fn[...]
    out[...] = jnp.dot(f, W_f1[...], preferred_element_type=jnp.float32) + b_f1[...]


def _full(shape):
    return pl.BlockSpec(shape, lambda i: (0,) * len(shape))


_tc_call = pl.pallas_call(
    _tc_body,
    grid=(GRID,),
    in_specs=[
        pl.BlockSpec((2, BN, HH), lambda i: (0, i, 0)),
        pl.BlockSpec((BN, D_IN), lambda i: (i, 0)),
        _full((1, H)),
        _full((1, H)), _full((1, H)),
        _full((H, H)), _full((1, H)),
        _full((D_IN, H)), _full((1, H)),
        _full((1, H)), _full((1, H)),
        _full((H, H)), _full((1, H)),
        _full((H, H)), _full((1, H)),
        _full((H, H)), _full((1, H)),
        _full((H, H)), _full((1, H)),
        _full((1, H)), _full((1, H)),
        _full((H, OUT)), _full((1, OUT)),
    ],
    out_specs=pl.BlockSpec((BN, OUT), lambda i: (i, 0)),
    out_shape=jax.ShapeDtypeStruct((N, OUT), jnp.float32),
)


def kernel(x, edge_index, W_edge, b_edge, g_en, be_en, W_em, b_em, W_n0,
           b_n0, g_nn, be_nn, W_n1, b_n1, W_c1, b_c1, W_c2, b_c2, W_f0,
           b_f0, g_fn, be_fn, W_f1, b_f1):
    src = edge_index[0]
    dst = edge_index[1]
    w2 = W_edge.reshape(2 * N, HH)
    src3 = src
    dst3 = dst.reshape(NTILES, NCHUNK, CHUNK)
    zrows = jnp.zeros((STRIPE, HH), jnp.float32)

    seg = _sc_call(w2, src3, dst3, zrows)

    r1 = lambda v: v.reshape(1, -1)
    return _tc_call(seg, x, r1(b_edge), r1(g_en), r1(be_en), W_em, r1(b_em),
                    W_n0, r1(b_n0), r1(g_nn), r1(be_nn), W_n1, r1(b_n1),
                    W_c1, r1(b_c1), W_c2, r1(b_c2), W_f0, r1(b_f0),
                    r1(g_fn), r1(be_fn), W_f1, r1(b_f1))


# speedup vs baseline: 3.7626x; 1.0072x over previous
import jax
import jax.numpy as jnp
from jax import lax
from jax.experimental import pallas as pl
from jax.experimental.pallas import tpu as pltpu
from jax.experimental.pallas import tpu_sc as plsc

N = 10000
E = 320000
D_IN = 128
H = 256
OUT = 64
EPS = 1e-5

HH = H // 2
NTILES = 16
CHUNK = 80
NCHUNK = E // NTILES // CHUNK
G = 50
PT = NCHUNK * CHUNK
STRIPE = N // NTILES
BN = 2000
GRID = N // BN


def _sc_body(w2, src3, dst3, zrows, out, acc, src_v, dst_v, rows0, rows1,
             rows2, rows3, sem0, sem1, sem2, sem3):
    rows = (rows0, rows1, rows2, rows3)
    sems = (sem0, sem1, sem2, sem3)
    c = lax.axis_index("c")
    s = lax.axis_index("s")
    pltpu.sync_copy(zrows, acc.at[pl.ds(s * STRIPE, STRIPE)])
    plsc.subcore_barrier()

    def group(g, carry):
        pltpu.sync_copy(src3.at[pl.ds(s * PT + g * G * CHUNK, G * CHUNK)],
                        src_v)
        pltpu.sync_copy(dst3.at[s, pl.ds(g * G, G)], dst_v)

        def xform(i, carry3):
            v = src_v[pl.ds(i * 16, 16)]
            src_v[pl.ds(i * 16, 16)] = v * 2 + c
            return carry3

        lax.fori_loop(0, G * CHUNK // 16, xform, 0)

        for b in range(4):
            pltpu.async_copy(
                w2.at[src_v.at[pl.ds(b * CHUNK, CHUNK)]], rows[b], sems[b])

        def quad(i, carry2):
            j0 = 4 * i
            for b in range(4):
                j = j0 + b
                pltpu.make_async_copy(
                    w2.at[src_v.at[pl.ds(j * CHUNK, CHUNK)]],
                    rows[b], sems[b]).wait()
                pltpu.sync_copy(rows[b], acc.at[dst_v.at[j]], add=True)

                @pl.when(j + 4 < G)
                def _():
                    pltpu.async_copy(
                        w2.at[src_v.at[pl.ds((j + 4) * CHUNK, CHUNK)]],
                        rows[b], sems[b])

            return carry2

        lax.fori_loop(0, G // 4, quad, 0)
        for b in range(G - 4 * (G // 4)):
            j = 4 * (G // 4) + b
            pltpu.make_async_copy(
                w2.at[src_v.at[pl.ds(j * CHUNK, CHUNK)]],
                rows[b], sems[b]).wait()
            pltpu.sync_copy(rows[b], acc.at[dst_v.at[j]], add=True)
        return carry

    lax.fori_loop(0, NCHUNK // G, group, 0)

    plsc.subcore_barrier()

    pltpu.sync_copy(
        acc.at[pl.ds(s * STRIPE, STRIPE)],
        out.at[c, pl.ds(s * STRIPE, STRIPE)],
    )


_sc_call = pl.kernel(
    _sc_body,
    out_type=jax.ShapeDtypeStruct((2, N, HH), jnp.float32),
    mesh=plsc.VectorSubcoreMesh(core_axis_name="c", subcore_axis_name="s"),
    scratch_types=[
        pltpu.VMEM_SHARED((N, HH), jnp.float32),
        pltpu.VMEM((G * CHUNK,), jnp.int32),
        pltpu.VMEM((G, CHUNK), jnp.int32),
        pltpu.VMEM((CHUNK, HH), jnp.float32),
        pltpu.VMEM((CHUNK, HH), jnp.float32),
        pltpu.VMEM((CHUNK, HH), jnp.float32),
        pltpu.VMEM((CHUNK, HH), jnp.float32),
        pltpu.SemaphoreType.DMA,
        pltpu.SemaphoreType.DMA,
        pltpu.SemaphoreType.DMA,
        pltpu.SemaphoreType.DMA,
    ],
    compiler_params=pltpu.CompilerParams(use_tc_tiling_on_sc=False),
)


def _tc_body(seg, x, b_edge, g_en, be_en, W_em, b_em, W_n0, b_n0, g_nn,
             be_nn, W_n1, b_n1, W_c1, b_c1, W_c2, b_c2, W_f0, b_f0, g_fn,
             be_fn, W_f1, b_f1, out):
    inv = 1.0 / jnp.sqrt(jnp.float32(1.0 + EPS))
    s = jnp.concatenate([seg[0], seg[1]], axis=-1) + b_edge[...]
    s = jnp.maximum(s, 0.0)
    s = s * (inv * g_en[...]) + be_en[...]
    o = jnp.dot(s, W_em[...], preferred_element_type=jnp.float32) + b_em[...]
    o = o + jnp.dot(o, W_c1[...], preferred_element_type=jnp.float32) + b_c1[...]
    h = jnp.dot(x[...], W_n0[...], preferred_element_type=jnp.float32) + b_n0[...]
    h = jnp.maximum(h, 0.0)
    h = h * (inv * g_nn[...]) + be_nn[...]
    h2 = jnp.dot(h, W_n1[...], preferred_element_type=jnp.float32) + b_n1[...]
    o = o + h2 + jnp.dot(h2, W_c2[...], preferred_element_type=jnp.float32) + b_c2[...]
    o = jnp.maximum(o, 0.0)
    f = jnp.dot(o, W_f0[...], preferred_element_type=jnp.float32) + b_f0[...]
    f = jnp.maximum(f, 0.0)
    f = f * (inv * g_fn[...]) + be_fn[...]
    out[...] = jnp.dot(f, W_f1[...], preferred_element_type=jnp.float32) + b_f1[...]


def _full(shape):
    return pl.BlockSpec(shape, lambda i: (0,) * len(shape))


_tc_call = pl.pallas_call(
    _tc_body,
    grid=(GRID,),
    in_specs=[
        pl.BlockSpec((2, BN, HH), lambda i: (0, i, 0)),
        pl.BlockSpec((BN, D_IN), lambda i: (i, 0)),
        _full((1, H)),
        _full((1, H)), _full((1, H)),
        _full((H, H)), _full((1, H)),
        _full((D_IN, H)), _full((1, H)),
        _full((1, H)), _full((1, H)),
        _full((H, H)), _full((1, H)),
        _full((H, H)), _full((1, H)),
        _full((H, H)), _full((1, H)),
        _full((H, H)), _full((1, H)),
        _full((1, H)), _full((1, H)),
        _full((H, OUT)), _full((1, OUT)),
    ],
    out_specs=pl.BlockSpec((BN, OUT), lambda i: (i, 0)),
    out_shape=jax.ShapeDtypeStruct((N, OUT), jnp.float32),
)


def kernel(x, edge_index, W_edge, b_edge, g_en, be_en, W_em, b_em, W_n0,
           b_n0, g_nn, be_nn, W_n1, b_n1, W_c1, b_c1, W_c2, b_c2, W_f0,
           b_f0, g_fn, be_fn, W_f1, b_f1):
    src = edge_index[0]
    dst = edge_index[1]
    w2 = W_edge.reshape(2 * N, HH)
    src3 = src
    dst3 = dst.reshape(NTILES, NCHUNK, CHUNK)
    zrows = jnp.zeros((STRIPE, HH), jnp.float32)

    seg = _sc_call(w2, src3, dst3, zrows)

    r1 = lambda v: v.reshape(1, -1)
    return _tc_call(seg, x, r1(b_edge), r1(g_en), r1(be_en), W_em, r1(b_em),
                    W_n0, r1(b_n0), r1(g_nn), r1(be_nn), W_n1, r1(b_n1),
                    W_c1, r1(b_c1), W_c2, r1(b_c2), W_f0, r1(b_f0),
                    r1(g_fn), r1(be_fn), W_f1, r1(b_f1))


# speedup vs baseline: 3.7653x; 1.0007x over previous
import jax
import jax.numpy as jnp
from jax import lax
from jax.experimental import pallas as pl
from jax.experimental.pallas import tpu as pltpu
from jax.experimental.pallas import tpu_sc as plsc

N = 10000
E = 320000
D_IN = 128
H = 256
OUT = 64
EPS = 1e-5

HH = H // 2
NTILES = 16
CHUNK = 80
NCHUNK = E // NTILES // CHUNK
G = 50
PT = NCHUNK * CHUNK
STRIPE = N // NTILES
BN = 2000
GRID = N // BN


def _sc_body(w2, src3, dst3, zrows, out, acc, src_v, dst_v, rows0, rows1,
             rows2, rows3, sem0, sem1, sem2, sem3):
    rows = (rows0, rows1, rows2, rows3)
    sems = (sem0, sem1, sem2, sem3)
    c = lax.axis_index("c")
    s = lax.axis_index("s")
    pltpu.sync_copy(zrows, acc.at[pl.ds(s * STRIPE, STRIPE)])
    plsc.subcore_barrier()

    def group(g, carry):
        pltpu.sync_copy(src3.at[pl.ds(s * PT + g * G * CHUNK, G * CHUNK)],
                        src_v)
        pltpu.sync_copy(dst3.at[pl.ds(s * PT + g * G * CHUNK, G * CHUNK)],
                        dst_v)

        def xform(i, carry3):
            v = src_v[pl.ds(i * 16, 16)]
            src_v[pl.ds(i * 16, 16)] = v * 2 + c
            return carry3

        lax.fori_loop(0, G * CHUNK // 16, xform, 0)

        for b in range(4):
            pltpu.async_copy(
                w2.at[src_v.at[pl.ds(b * CHUNK, CHUNK)]], rows[b], sems[b])

        def quad(i, carry2):
            j0 = 4 * i
            for b in range(4):
                j = j0 + b
                pltpu.make_async_copy(
                    w2.at[src_v.at[pl.ds(j * CHUNK, CHUNK)]],
                    rows[b], sems[b]).wait()
                pltpu.sync_copy(rows[b], acc.at[dst_v.at[pl.ds(j * CHUNK, CHUNK)]], add=True)

                @pl.when(j + 4 < G)
                def _():
                    pltpu.async_copy(
                        w2.at[src_v.at[pl.ds((j + 4) * CHUNK, CHUNK)]],
                        rows[b], sems[b])

            return carry2

        lax.fori_loop(0, G // 4, quad, 0)
        for b in range(G - 4 * (G // 4)):
            j = 4 * (G // 4) + b
            pltpu.make_async_copy(
                w2.at[src_v.at[pl.ds(j * CHUNK, CHUNK)]],
                rows[b], sems[b]).wait()
            pltpu.sync_copy(rows[b], acc.at[dst_v.at[pl.ds(j * CHUNK, CHUNK)]], add=True)
        return carry

    lax.fori_loop(0, NCHUNK // G, group, 0)

    plsc.subcore_barrier()

    pltpu.sync_copy(
        acc.at[pl.ds(s * STRIPE, STRIPE)],
        out.at[c, pl.ds(s * STRIPE, STRIPE)],
    )


_sc_call = pl.kernel(
    _sc_body,
    out_type=jax.ShapeDtypeStruct((2, N, HH), jnp.float32),
    mesh=plsc.VectorSubcoreMesh(core_axis_name="c", subcore_axis_name="s"),
    scratch_types=[
        pltpu.VMEM_SHARED((N, HH), jnp.float32),
        pltpu.VMEM((G * CHUNK,), jnp.int32),
        pltpu.VMEM((G * CHUNK,), jnp.int32),
        pltpu.VMEM((CHUNK, HH), jnp.float32),
        pltpu.VMEM((CHUNK, HH), jnp.float32),
        pltpu.VMEM((CHUNK, HH), jnp.float32),
        pltpu.VMEM((CHUNK, HH), jnp.float32),
        pltpu.SemaphoreType.DMA,
        pltpu.SemaphoreType.DMA,
        pltpu.SemaphoreType.DMA,
        pltpu.SemaphoreType.DMA,
    ],
    compiler_params=pltpu.CompilerParams(use_tc_tiling_on_sc=False),
)


def _tc_body(seg, x, b_edge, g_en, be_en, W_em, b_em, W_n0, b_n0, g_nn,
             be_nn, W_n1, b_n1, W_c1, b_c1, W_c2, b_c2, W_f0, b_f0, g_fn,
             be_fn, W_f1, b_f1, out):
    inv = 1.0 / jnp.sqrt(jnp.float32(1.0 + EPS))
    s = jnp.concatenate([seg[0], seg[1]], axis=-1) + b_edge[...]
    s = jnp.maximum(s, 0.0)
    s = s * (inv * g_en[...]) + be_en[...]
    o = jnp.dot(s, W_em[...], preferred_element_type=jnp.float32) + b_em[...]
    o = o + jnp.dot(o, W_c1[...], preferred_element_type=jnp.float32) + b_c1[...]
    h = jnp.dot(x[...], W_n0[...], preferred_element_type=jnp.float32) + b_n0[...]
    h = jnp.maximum(h, 0.0)
    h = h * (inv * g_nn[...]) + be_nn[...]
    h2 = jnp.dot(h, W_n1[...], preferred_element_type=jnp.float32) + b_n1[...]
    o = o + h2 + jnp.dot(h2, W_c2[...], preferred_element_type=jnp.float32) + b_c2[...]
    o = jnp.maximum(o, 0.0)
    f = jnp.dot(o, W_f0[...], preferred_element_type=jnp.float32) + b_f0[...]
    f = jnp.maximum(f, 0.0)
    f = f * (inv * g_fn[...]) + be_fn[...]
    out[...] = jnp.dot(f, W_f1[...], preferred_element_type=jnp.float32) + b_f1[...]


def _full(shape):
    return pl.BlockSpec(shape, lambda i: (0,) * len(shape))


_tc_call = pl.pallas_call(
    _tc_body,
    grid=(GRID,),
    in_specs=[
        pl.BlockSpec((2, BN, HH), lambda i: (0, i, 0)),
        pl.BlockSpec((BN, D_IN), lambda i: (i, 0)),
        _full((1, H)),
        _full((1, H)), _full((1, H)),
        _full((H, H)), _full((1, H)),
        _full((D_IN, H)), _full((1, H)),
        _full((1, H)), _full((1, H)),
        _full((H, H)), _full((1, H)),
        _full((H, H)), _full((1, H)),
        _full((H, H)), _full((1, H)),
        _full((H, H)), _full((1, H)),
        _full((1, H)), _full((1, H)),
        _full((H, OUT)), _full((1, OUT)),
    ],
    out_specs=pl.BlockSpec((BN, OUT), lambda i: (i, 0)),
    out_shape=jax.ShapeDtypeStruct((N, OUT), jnp.float32),
)


def kernel(x, edge_index, W_edge, b_edge, g_en, be_en, W_em, b_em, W_n0,
           b_n0, g_nn, be_nn, W_n1, b_n1, W_c1, b_c1, W_c2, b_c2, W_f0,
           b_f0, g_fn, be_fn, W_f1, b_f1):
    src = edge_index[0]
    dst = edge_index[1]
    w2 = W_edge.reshape(2 * N, HH)
    src3 = src
    dst3 = dst
    zrows = jnp.zeros((STRIPE, HH), jnp.float32)

    seg = _sc_call(w2, src3, dst3, zrows)

    r1 = lambda v: v.reshape(1, -1)
    return _tc_call(seg, x, r1(b_edge), r1(g_en), r1(be_en), W_em, r1(b_em),
                    W_n0, r1(b_n0), r1(g_nn), r1(be_nn), W_n1, r1(b_n1),
                    W_c1, r1(b_c1), W_c2, r1(b_c2), W_f0, r1(b_f0),
                    r1(g_fn), r1(be_fn), W_f1, r1(b_f1))
